# baseline (speedup 1.0000x reference)
import functools

import jax
import jax.numpy as jnp
from jax import lax
from jax.experimental import pallas as pl
from jax.experimental.pallas import tpu as pltpu
from jax.experimental.pallas import tpu_sc as plsc

_NEG_INF = float("-inf")

N = 4096
D = 64
E = 131072

_NC = 2
_NS = 16
_NW = _NC * _NS
_EPT = E // _NW
_IDX_ROWS = 2 * _EPT // 128


def _scatter_body(edges_hbm, mask_ref, src_v, dst_v, idx_v, ones_v, sem):
    c = lax.axis_index("c")
    s = lax.axis_index("s")
    wid = s * _NC + c
    base = wid * _EPT
    pltpu.sync_copy(edges_hbm.at[pl.ds(base, _EPT)], src_v)
    pltpu.sync_copy(edges_hbm.at[pl.ds(E + base, _EPT)], dst_v)

    @pl.loop(0, 2 * _EPT // 16)
    def _fill_ones(k):
        ones_v[pl.ds(k * 16, 16)] = jnp.ones((16,), jnp.float32)

    @pl.loop(0, _EPT // 16)
    def _build(i):
        sv = src_v[pl.ds(i * 16, 16)]
        dv = dst_v[pl.ds(i * 16, 16)]
        f1 = sv * N + dv
        f2 = dv * N + sv
        idx_v[pl.ds(i * 32, 16)] = f1
        idx_v[pl.ds(i * 32 + 16, 16)] = f2

    pltpu.async_copy(ones_v, mask_ref.at[idx_v], sem).wait()


def _build_mask(edges_flat, mask_ref):
    mesh = plsc.VectorSubcoreMesh(core_axis_name="c", subcore_axis_name="s")
    f = pl.kernel(
        _scatter_body,
        out_type=(),
        mesh=mesh,
        scratch_types=[
            pltpu.VMEM((_EPT,), jnp.int32),
            pltpu.VMEM((_EPT,), jnp.int32),
            pltpu.VMEM((2 * _EPT,), jnp.int32),
            pltpu.VMEM((2 * _EPT,), jnp.float32),
            pltpu.SemaphoreType.DMA,
        ],
    )
    f(edges_flat, mask_ref)


def _attn_body(mask_ref, hb_ref, hall_ref, aw_ref, out_ref):
    hb = hb_ref[...]
    hall = hall_ref[...]
    aw = aw_ref[...]
    q = hb * aw
    s = jax.lax.dot_general(
        q, hall, (((1,), (1,)), ((), ())),
        preferred_element_type=jnp.float32)
    s = jnp.where(s > 0, s, 0.2 * s)
    mask = mask_ref[...] > 0
    sm = jnp.where(mask, s, _NEG_INF)
    m = jnp.max(sm, axis=1, keepdims=True)
    m = jnp.where(jnp.isfinite(m), m, 0.0)
    e = jnp.where(mask, jnp.exp(s - m), 0.0)
    den = jnp.sum(e, axis=1, keepdims=True)
    alpha = e / jnp.where(den > 0, den, 1.0)
    out_ref[...] = jax.lax.dot_general(
        alpha, hall, (((1,), (0,)), ((), ())),
        preferred_element_type=jnp.float32)


def _attention(mask, hidden, a_row):
    n, d = hidden.shape
    br = 256
    grid = (n // br,)
    return pl.pallas_call(
        _attn_body,
        grid=grid,
        in_specs=[
            pl.BlockSpec((br, n), lambda i: (i, 0)),
            pl.BlockSpec((br, d), lambda i: (i, 0)),
            pl.BlockSpec((n, d), lambda i: (0, 0)),
            pl.BlockSpec((1, d), lambda i: (0, 0)),
        ],
        out_specs=pl.BlockSpec((br, d), lambda i: (i, 0)),
        out_shape=jax.ShapeDtypeStruct((n, d), jnp.float32),
    )(mask, hidden, hidden, a_row)


@jax.jit
def _run(hidden, edge_index, a_w):
    n, d = hidden.shape
    edges_flat = edge_index.reshape(-1).astype(jnp.int32)
    mask_ref = jax.new_ref(jnp.zeros((n * n,), jnp.float32))
    _build_mask(edges_flat, mask_ref)
    mask = mask_ref[...].reshape(n, n)
    a_row = a_w.reshape(1, d)
    return _attention(mask, hidden, a_row)


def kernel(hidden, edge_index, batch, a_w):
    return _run(hidden, edge_index, a_w)


# speedup vs baseline: 1.0079x; 1.0079x over previous
import functools

import jax
import jax.numpy as jnp
from jax import lax
from jax.experimental import pallas as pl
from jax.experimental.pallas import tpu as pltpu
from jax.experimental.pallas import tpu_sc as plsc

_NEG_INF = float("-inf")

N = 4096
D = 64
E = 131072

_NC = 2
_NS = 16
_NW = _NC * _NS
_EPT = E // _NW
_IDX_ROWS = 2 * _EPT // 128


def _scatter_body(edges_hbm, mask_ref, src_v, dst_v, idx_v, ones_v, sem):
    c = lax.axis_index("c")
    s = lax.axis_index("s")
    wid = s * _NC + c
    base = wid * _EPT
    pltpu.sync_copy(edges_hbm.at[pl.ds(base, _EPT)], src_v)
    pltpu.sync_copy(edges_hbm.at[pl.ds(E + base, _EPT)], dst_v)

    @pl.loop(0, 2 * _EPT // 16)
    def _fill_ones(k):
        ones_v[pl.ds(k * 16, 16)] = jnp.ones((16,), jnp.float32)

    @pl.loop(0, _EPT // 16)
    def _build(i):
        sv = src_v[pl.ds(i * 16, 16)]
        dv = dst_v[pl.ds(i * 16, 16)]
        f1 = sv * N + dv
        f2 = dv * N + sv
        idx_v[pl.ds(i * 32, 16)] = f1
        idx_v[pl.ds(i * 32 + 16, 16)] = f2

    pltpu.async_copy(ones_v, mask_ref.at[idx_v], sem).wait()


def _build_mask(edges_flat, mask_ref):
    mesh = plsc.VectorSubcoreMesh(core_axis_name="c", subcore_axis_name="s")
    f = pl.kernel(
        _scatter_body,
        out_type=(),
        mesh=mesh,
        scratch_types=[
            pltpu.VMEM((_EPT,), jnp.int32),
            pltpu.VMEM((_EPT,), jnp.int32),
            pltpu.VMEM((2 * _EPT,), jnp.int32),
            pltpu.VMEM((2 * _EPT,), jnp.float32),
            pltpu.SemaphoreType.DMA,
        ],
    )
    f(edges_flat, mask_ref)


def _attn_body(mask_ref, hb_ref, hall_ref, aw_ref, out_ref):
    big = jnp.float32(1e38)
    hb = hb_ref[...]
    hall = hall_ref[...]
    aw = aw_ref[...]
    q = hb * aw
    s = jax.lax.dot_general(
        q, hall, (((1,), (1,)), ((), ())),
        preferred_element_type=jnp.float32)
    s = jnp.maximum(s, 0.2 * s)
    sm = s + (mask_ref[...] * big - big)
    m = jnp.max(sm, axis=1, keepdims=True)
    has = m > jnp.float32(-1e37)
    mc = jnp.where(has, m, 0.0)
    e = jnp.exp(sm - mc)
    den = jnp.sum(e, axis=1, keepdims=True)
    alpha = e * jnp.where(has, 1.0 / den, 0.0)
    out_ref[...] = jax.lax.dot_general(
        alpha, hall, (((1,), (0,)), ((), ())),
        preferred_element_type=jnp.float32)


def _attention(mask, hidden, a_row):
    n, d = hidden.shape
    br = 256
    grid = (n // br,)
    return pl.pallas_call(
        _attn_body,
        grid=grid,
        in_specs=[
            pl.BlockSpec((br, n), lambda i: (i, 0)),
            pl.BlockSpec((br, d), lambda i: (i, 0)),
            pl.BlockSpec((n, d), lambda i: (0, 0)),
            pl.BlockSpec((1, d), lambda i: (0, 0)),
        ],
        out_specs=pl.BlockSpec((br, d), lambda i: (i, 0)),
        out_shape=jax.ShapeDtypeStruct((n, d), jnp.float32),
    )(mask, hidden, hidden, a_row)


@jax.jit
def _run(hidden, edge_index, a_w):
    n, d = hidden.shape
    edges_flat = edge_index.reshape(-1).astype(jnp.int32)
    mask_ref = jax.new_ref(jnp.zeros((n * n,), jnp.float32))
    _build_mask(edges_flat, mask_ref)
    mask = mask_ref[...].reshape(n, n)
    a_row = a_w.reshape(1, d)
    return _attention(mask, hidden, a_row)


def kernel(hidden, edge_index, batch, a_w):
    return _run(hidden, edge_index, a_w)


# speedup vs baseline: 1.3846x; 1.3738x over previous
import jax
import jax.numpy as jnp
from jax import lax
from jax.experimental import pallas as pl
from jax.experimental.pallas import tpu as pltpu
from jax.experimental.pallas import tpu_sc as plsc

N = 4096
D = 64
E = 131072

_NC = 2
_NS = 16
_NW = _NC * _NS
_EPT = E // _NW


def _scatter_body(edges_hbm, ones_hbm, mask_ref, src_v, dst_v, idx_v, ones_v,
                  sem):
    c = lax.axis_index("c")
    s = lax.axis_index("s")
    wid = s * _NC + c
    base = wid * _EPT
    pltpu.sync_copy(edges_hbm.at[pl.ds(base, _EPT)], src_v)
    pltpu.sync_copy(edges_hbm.at[pl.ds(E + base, _EPT)], dst_v)
    pltpu.sync_copy(ones_hbm, ones_v)

    @pl.loop(0, _EPT // 16, unroll=8)
    def _build(i):
        sv = src_v[pl.ds(i * 16, 16)]
        dv = dst_v[pl.ds(i * 16, 16)]
        idx_v[pl.ds(i * 16, 16)] = sv * N + dv

    pltpu.async_copy(ones_v, mask_ref.at[idx_v], sem).wait()


def _build_mask(edges_flat, ones, mask_ref):
    mesh = plsc.VectorSubcoreMesh(core_axis_name="c", subcore_axis_name="s")
    f = pl.kernel(
        _scatter_body,
        out_type=(),
        mesh=mesh,
        scratch_types=[
            pltpu.VMEM((_EPT,), jnp.int32),
            pltpu.VMEM((_EPT,), jnp.int32),
            pltpu.VMEM((_EPT,), jnp.int32),
            pltpu.VMEM((_EPT,), jnp.float32),
            pltpu.SemaphoreType.DMA,
        ],
    )
    f(edges_flat, ones, mask_ref)


def _attn_body(mask_ref, maskc_ref, hb_ref, hall_ref, aw_ref, ident_ref,
               out_ref):
    big = jnp.float32(1e38)
    hb = hb_ref[...]
    hall = hall_ref[...]
    aw = aw_ref[...]
    q = hb * aw
    s = lax.dot_general(
        q, hall, (((1,), (1,)), ((), ())),
        preferred_element_type=jnp.float32)
    s = jnp.maximum(s, 0.2 * s)
    mt = lax.dot_general(
        ident_ref[...], maskc_ref[...], (((1,), (1,)), ((), ())),
        preferred_element_type=jnp.float32)
    union = jnp.minimum(mask_ref[...] + mt, 1.0)
    sm = s + (union * big - big)
    m = jnp.max(sm, axis=1, keepdims=True)
    has = m > jnp.float32(-1e37)
    mc = jnp.where(has, m, 0.0)
    e = jnp.exp(sm - mc)
    den = jnp.sum(e, axis=1, keepdims=True)
    alpha = e * jnp.where(has, 1.0 / den, 0.0)
    out_ref[...] = lax.dot_general(
        alpha, hall, (((1,), (0,)), ((), ())),
        preferred_element_type=jnp.float32)


def _attention(mask, hidden, a_row, ident):
    n, d = hidden.shape
    br = 256
    grid = (n // br,)
    return pl.pallas_call(
        _attn_body,
        grid=grid,
        in_specs=[
            pl.BlockSpec((br, n), lambda i: (i, 0)),
            pl.BlockSpec((n, br), lambda i: (0, i)),
            pl.BlockSpec((br, d), lambda i: (i, 0)),
            pl.BlockSpec((n, d), lambda i: (0, 0)),
            pl.BlockSpec((1, d), lambda i: (0, 0)),
            pl.BlockSpec((br, br), lambda i: (0, 0)),
        ],
        out_specs=pl.BlockSpec((br, d), lambda i: (i, 0)),
        out_shape=jax.ShapeDtypeStruct((n, d), jnp.float32),
    )(mask, mask, hidden, hidden, a_row, ident)


@jax.jit
def _run(hidden, edge_index, a_w):
    n, d = hidden.shape
    edges_flat = edge_index.reshape(-1).astype(jnp.int32)
    ones = jnp.ones((_EPT,), jnp.float32)
    mask_ref = jax.new_ref(jnp.zeros((n * n,), jnp.float32))
    _build_mask(edges_flat, ones, mask_ref)
    mask = mask_ref[...].reshape(n, n)
    a_row = a_w.reshape(1, d)
    ident = jnp.eye(256, dtype=jnp.float32)
    return _attention(mask, hidden, a_row, ident)


def kernel(hidden, edge_index, batch, a_w):
    return _run(hidden, edge_index, a_w)
